# Initial kernel scaffold; baseline (speedup 1.0000x reference)
import functools

import jax
import jax.numpy as jnp
from jax import lax
from jax.experimental import pallas as pl
from jax.experimental.pallas import tpu as pltpu
from jax.experimental.pallas import tpu_sc as plsc

BATCH = 16384
L = 20
EMBED = 64
HIDDEN = 128

NC = 2
NS = 16
NW = NC * NS

TASKS = 2 * BATCH
TASKS_PER_W = TASKS // NW
CHUNK = 32
NCHUNK = TASKS_PER_W // CHUNK
ROWS_PER_CHUNK = CHUNK * L
GATHER_SLICE = 128
NSLICE = ROWS_PER_CHUNK // GATHER_SLICE


def _pool_kernel(idx_hbm, table_hbm, out_hbm, idx_v, rows_v, out_v, sem):
    wid = lax.axis_index("s") * NC + lax.axis_index("c")

    def chunk_body(c, _):
        base_task = wid * TASKS_PER_W + c * CHUNK
        idx_off = pl.multiple_of(base_task * L, ROWS_PER_CHUNK)
        pltpu.sync_copy(idx_hbm.at[pl.ds(idx_off, ROWS_PER_CHUNK)], idx_v)
        copies = [
            pltpu.async_copy(
                table_hbm.at[idx_v.at[pl.ds(j * GATHER_SLICE, GATHER_SLICE)]],
                rows_v.at[pl.ds(j * GATHER_SLICE, GATHER_SLICE)],
                sem,
            )
            for j in range(NSLICE)
        ]
        for cp in copies:
            cp.wait()

        def task_body(t, _):
            for g in range(EMBED // 16):
                acc = rows_v[t * L, pl.ds(g * 16, 16)]
                for r in range(1, L):
                    acc = acc + rows_v[t * L + r, pl.ds(g * 16, 16)]
                out_v[t, pl.ds(g * 16, 16)] = acc * (1.0 / L)
            return 0

        lax.fori_loop(0, CHUNK, task_body, 0)
        pltpu.sync_copy(out_v, out_hbm.at[pl.ds(base_task, CHUNK)])
        return 0

    lax.fori_loop(0, NCHUNK, chunk_body, 0)


@functools.partial(
    pl.kernel,
    mesh=plsc.VectorSubcoreMesh(core_axis_name="c", subcore_axis_name="s"),
    out_type=jax.ShapeDtypeStruct((TASKS, EMBED), jnp.float32),
    scratch_types=[
        pltpu.VMEM((ROWS_PER_CHUNK,), jnp.int32),
        pltpu.VMEM((ROWS_PER_CHUNK, EMBED), jnp.float32),
        pltpu.VMEM((CHUNK, EMBED), jnp.float32),
        pltpu.SemaphoreType.DMA,
    ],
)
def _pool(idx_hbm, table_hbm, out_hbm, idx_v, rows_v, out_v, sem):
    _pool_kernel(idx_hbm, table_hbm, out_hbm, idx_v, rows_v, out_v, sem)


MLP_TILE = 512


def _mlp_body(xa_ref, xb_ref, w1a_ref, w1b_ref, b1_ref, w2_ref, b2_ref,
              w3_ref, b3_ref, out_ref):
    xa = xa_ref[...]
    xb = xb_ref[...]
    h = (jnp.dot(xa, w1a_ref[...], preferred_element_type=jnp.float32)
         + jnp.dot(xb, w1b_ref[...], preferred_element_type=jnp.float32)
         + b1_ref[...])
    h = jnp.maximum(h, 0.0)
    h = jnp.dot(h, w2_ref[...], preferred_element_type=jnp.float32) + b2_ref[...]
    h = jnp.maximum(h, 0.0)
    logit = jnp.sum(h * w3_ref[...], axis=1) + b3_ref[0, 0]
    out_ref[0, :] = jax.nn.sigmoid(logit)


def _mlp(pooled, w1t, b1, w2t, b2, w3, b3):
    grid = (BATCH // MLP_TILE,)
    full = lambda i: (0, 0)
    out = pl.pallas_call(
        _mlp_body,
        grid=grid,
        in_specs=[
            pl.BlockSpec((MLP_TILE, EMBED), lambda i: (i, 0)),
            pl.BlockSpec((MLP_TILE, EMBED), lambda i: (i + BATCH // MLP_TILE, 0)),
            pl.BlockSpec((EMBED, HIDDEN), full),
            pl.BlockSpec((EMBED, HIDDEN), full),
            pl.BlockSpec((1, HIDDEN), full),
            pl.BlockSpec((HIDDEN, HIDDEN), full),
            pl.BlockSpec((1, HIDDEN), full),
            pl.BlockSpec((1, HIDDEN), full),
            pl.BlockSpec((1, 1), full),
        ],
        out_specs=pl.BlockSpec((1, MLP_TILE), lambda i: (0, i)),
        out_shape=jax.ShapeDtypeStruct((1, BATCH), jnp.float32),
    )(pooled, pooled, w1t[:EMBED], w1t[EMBED:], b1.reshape(1, HIDDEN),
      w2t, b2.reshape(1, HIDDEN), w3.reshape(1, HIDDEN), b3.reshape(1, 1))
    return out[0]


def kernel(a_indices_list, b_indices_list, table, W1, b1, W2, b2, W3, b3):
    idx = jnp.concatenate(
        [a_indices_list.reshape(-1), b_indices_list.reshape(-1)]
    ).astype(jnp.int32)
    pooled = _pool(idx, table)
    return _mlp(pooled, W1.T, b1, W2.T, b2, W3, b3)


# speedup vs baseline: 1.1558x; 1.1558x over previous
import functools

import jax
import jax.numpy as jnp
from jax import lax
from jax.experimental import pallas as pl
from jax.experimental.pallas import tpu as pltpu
from jax.experimental.pallas import tpu_sc as plsc

BATCH = 16384
L = 20
EMBED = 64
HIDDEN = 128

NC = 2
NS = 16
NW = NC * NS

TASKS = 2 * BATCH
TASKS_PER_W = TASKS // NW
CHUNK = 32
NCHUNK = TASKS_PER_W // CHUNK
ROWS_PER_CHUNK = CHUNK * L
GATHER_SLICE = 128
NSLICE = ROWS_PER_CHUNK // GATHER_SLICE


def _pool_kernel(idx_hbm, table_hbm, out_hbm, idx_v, rows_v, out_v, sem):
    wid = lax.axis_index("s") * NC + lax.axis_index("c")

    def chunk_body(c, _):
        base_task = wid * TASKS_PER_W + c * CHUNK
        idx_off = pl.multiple_of(base_task * L, ROWS_PER_CHUNK)
        pltpu.sync_copy(idx_hbm.at[pl.ds(idx_off, ROWS_PER_CHUNK)], idx_v)
        copies = [
            pltpu.async_copy(
                table_hbm.at[idx_v.at[pl.ds(j * GATHER_SLICE, GATHER_SLICE)]],
                rows_v.at[pl.ds(j * GATHER_SLICE, GATHER_SLICE)],
                sem,
            )
            for j in range(NSLICE)
        ]
        for cp in copies:
            cp.wait()

        def task_body(t, _):
            for g in range(EMBED // 16):
                acc = rows_v[t * L, pl.ds(g * 16, 16)]
                for r in range(1, L):
                    acc = acc + rows_v[t * L + r, pl.ds(g * 16, 16)]
                out_v[t, pl.ds(g * 16, 16)] = acc * (1.0 / L)
            return 0

        lax.fori_loop(0, CHUNK, task_body, 0)
        pltpu.sync_copy(out_v, out_hbm.at[pl.ds(base_task, CHUNK)])
        return 0

    lax.fori_loop(0, NCHUNK, chunk_body, 0)


@functools.partial(
    pl.kernel,
    mesh=plsc.VectorSubcoreMesh(core_axis_name="c", subcore_axis_name="s"),
    out_type=jax.ShapeDtypeStruct((TASKS, EMBED), jnp.float32),
    compiler_params=pltpu.CompilerParams(use_tc_tiling_on_sc=False),
    scratch_types=[
        pltpu.VMEM((ROWS_PER_CHUNK,), jnp.int32),
        pltpu.VMEM((ROWS_PER_CHUNK, EMBED), jnp.float32),
        pltpu.VMEM((CHUNK, EMBED), jnp.float32),
        pltpu.SemaphoreType.DMA,
    ],
)
def _pool(idx_hbm, table_hbm, out_hbm, idx_v, rows_v, out_v, sem):
    _pool_kernel(idx_hbm, table_hbm, out_hbm, idx_v, rows_v, out_v, sem)


MLP_TILE = 512


def _mlp_body(xa_ref, xb_ref, w1a_ref, w1b_ref, b1_ref, w2_ref, b2_ref,
              w3_ref, b3_ref, out_ref):
    xa = xa_ref[...]
    xb = xb_ref[...]
    h = (jnp.dot(xa, w1a_ref[...], preferred_element_type=jnp.float32)
         + jnp.dot(xb, w1b_ref[...], preferred_element_type=jnp.float32)
         + b1_ref[...])
    h = jnp.maximum(h, 0.0)
    h = jnp.dot(h, w2_ref[...], preferred_element_type=jnp.float32) + b2_ref[...]
    h = jnp.maximum(h, 0.0)
    logit = jnp.sum(h * w3_ref[...], axis=1) + b3_ref[0, 0]
    out_ref[0, :] = jax.nn.sigmoid(logit)


def _mlp(pooled, w1t, b1, w2t, b2, w3, b3):
    grid = (BATCH // MLP_TILE,)
    full = lambda i: (0, 0)
    out = pl.pallas_call(
        _mlp_body,
        grid=grid,
        in_specs=[
            pl.BlockSpec((MLP_TILE, EMBED), lambda i: (i, 0)),
            pl.BlockSpec((MLP_TILE, EMBED), lambda i: (i + BATCH // MLP_TILE, 0)),
            pl.BlockSpec((EMBED, HIDDEN), full),
            pl.BlockSpec((EMBED, HIDDEN), full),
            pl.BlockSpec((1, HIDDEN), full),
            pl.BlockSpec((HIDDEN, HIDDEN), full),
            pl.BlockSpec((1, HIDDEN), full),
            pl.BlockSpec((1, HIDDEN), full),
            pl.BlockSpec((1, 1), full),
        ],
        out_specs=pl.BlockSpec((1, MLP_TILE), lambda i: (0, i)),
        out_shape=jax.ShapeDtypeStruct((1, BATCH), jnp.float32),
    )(pooled, pooled, w1t[:EMBED], w1t[EMBED:], b1.reshape(1, HIDDEN),
      w2t, b2.reshape(1, HIDDEN), w3.reshape(1, HIDDEN), b3.reshape(1, 1))
    return out[0]


def kernel(a_indices_list, b_indices_list, table, W1, b1, W2, b2, W3, b3):
    idx = jnp.concatenate(
        [a_indices_list.reshape(-1), b_indices_list.reshape(-1)]
    ).astype(jnp.int32)
    pooled = _pool(idx, table)
    return _mlp(pooled, W1.T, b1, W2.T, b2, W3, b3)


# speedup vs baseline: 1.1570x; 1.0011x over previous
import functools

import jax
import jax.numpy as jnp
from jax import lax
from jax.experimental import pallas as pl
from jax.experimental.pallas import tpu as pltpu
from jax.experimental.pallas import tpu_sc as plsc

BATCH = 16384
L = 20
EMBED = 64
HIDDEN = 128

NC = 2
NS = 16
NW = NC * NS

TASKS = 2 * BATCH
TASKS_PER_SRC_W = BATCH // NW
CHUNK = 32
NCHUNK = TASKS_PER_SRC_W // CHUNK
ROWS_PER_CHUNK = CHUNK * L
GATHER_SLICE = 128
NSLICE = ROWS_PER_CHUNK // GATHER_SLICE


def _pool_kernel(a_hbm, b_hbm, table_hbm, out_hbm, idx_v, rows_v, out_v, sem):
    wid = lax.axis_index("s") * NC + lax.axis_index("c")

    for src_hbm, out_base in ((a_hbm, 0), (b_hbm, BATCH)):
        def chunk_body(c, _):
            task0 = wid * TASKS_PER_SRC_W + c * CHUNK
            idx_off = pl.multiple_of(task0 * L, ROWS_PER_CHUNK)
            pltpu.sync_copy(src_hbm.at[pl.ds(idx_off, ROWS_PER_CHUNK)], idx_v)
            copies = [
                pltpu.async_copy(
                    table_hbm.at[idx_v.at[pl.ds(j * GATHER_SLICE, GATHER_SLICE)]],
                    rows_v.at[pl.ds(j * GATHER_SLICE, GATHER_SLICE)],
                    sem,
                )
                for j in range(NSLICE)
            ]
            for cp in copies:
                cp.wait()

            def task_body(t, _):
                for g in range(EMBED // 16):
                    acc = rows_v[t * L, pl.ds(g * 16, 16)]
                    for r in range(1, L):
                        acc = acc + rows_v[t * L + r, pl.ds(g * 16, 16)]
                    out_v[t, pl.ds(g * 16, 16)] = acc * (1.0 / L)
                return 0

            lax.fori_loop(0, CHUNK, task_body, 0)
            pltpu.sync_copy(out_v, out_hbm.at[pl.ds(out_base + task0, CHUNK)])
            return 0

        lax.fori_loop(0, NCHUNK, chunk_body, 0)


@functools.partial(
    pl.kernel,
    mesh=plsc.VectorSubcoreMesh(core_axis_name="c", subcore_axis_name="s"),
    out_type=jax.ShapeDtypeStruct((TASKS, EMBED), jnp.float32),
    compiler_params=pltpu.CompilerParams(use_tc_tiling_on_sc=False),
    scratch_types=[
        pltpu.VMEM((ROWS_PER_CHUNK,), jnp.int32),
        pltpu.VMEM((ROWS_PER_CHUNK, EMBED), jnp.float32),
        pltpu.VMEM((CHUNK, EMBED), jnp.float32),
        pltpu.SemaphoreType.DMA,
    ],
)
def _pool(a_hbm, b_hbm, table_hbm, out_hbm, idx_v, rows_v, out_v, sem):
    _pool_kernel(a_hbm, b_hbm, table_hbm, out_hbm, idx_v, rows_v, out_v, sem)


MLP_TILE = 512


def _mlp_body(xa_ref, xb_ref, w1a_ref, w1b_ref, b1_ref, w2_ref, b2_ref,
              w3_ref, b3_ref, out_ref):
    xa = xa_ref[...]
    xb = xb_ref[...]
    h = (jnp.dot(xa, w1a_ref[...], preferred_element_type=jnp.float32)
         + jnp.dot(xb, w1b_ref[...], preferred_element_type=jnp.float32)
         + b1_ref[...])
    h = jnp.maximum(h, 0.0)
    h = jnp.dot(h, w2_ref[...], preferred_element_type=jnp.float32) + b2_ref[...]
    h = jnp.maximum(h, 0.0)
    logit = jnp.sum(h * w3_ref[...], axis=1) + b3_ref[0, 0]
    out_ref[0, :] = jax.nn.sigmoid(logit)


def _mlp(pooled, w1t, b1, w2t, b2, w3, b3):
    grid = (BATCH // MLP_TILE,)
    full = lambda i: (0, 0)
    out = pl.pallas_call(
        _mlp_body,
        grid=grid,
        in_specs=[
            pl.BlockSpec((MLP_TILE, EMBED), lambda i: (i, 0)),
            pl.BlockSpec((MLP_TILE, EMBED), lambda i: (i + BATCH // MLP_TILE, 0)),
            pl.BlockSpec((EMBED, HIDDEN), full),
            pl.BlockSpec((EMBED, HIDDEN), full),
            pl.BlockSpec((1, HIDDEN), full),
            pl.BlockSpec((HIDDEN, HIDDEN), full),
            pl.BlockSpec((1, HIDDEN), full),
            pl.BlockSpec((1, HIDDEN), full),
            pl.BlockSpec((1, 1), full),
        ],
        out_specs=pl.BlockSpec((1, MLP_TILE), lambda i: (0, i)),
        out_shape=jax.ShapeDtypeStruct((1, BATCH), jnp.float32),
    )(pooled, pooled, w1t[:EMBED], w1t[EMBED:], b1.reshape(1, HIDDEN),
      w2t, b2.reshape(1, HIDDEN), w3.reshape(1, HIDDEN), b3.reshape(1, 1))
    return out[0]


def kernel(a_indices_list, b_indices_list, table, W1, b1, W2, b2, W3, b3):
    a_idx = a_indices_list.reshape(-1).astype(jnp.int32)
    b_idx = b_indices_list.reshape(-1).astype(jnp.int32)
    pooled = _pool(a_idx, b_idx, table)
    return _mlp(pooled, W1.T, b1, W2.T, b2, W3, b3)
